# baseline (speedup 1.0000x reference)
import functools

import jax
import jax.numpy as jnp
from jax import lax
from jax.experimental import pallas as pl
from jax.experimental.pallas import tpu as pltpu
from jax.experimental.pallas import tpu_sc as plsc

_VOCAB_SIZE = 1000000
_MASK_TOKEN = -1
_PAD_TOKEN = -2
_LANES = 16


def _build_lookup(T, D, C):
    info = plsc.get_sparse_core_info()
    NC, NS = info.num_cores, info.num_subcores
    NW = NC * NS
    per_w = T // NW
    n_chunks = per_w // C
    assert n_chunks % 2 == 0
    mesh = plsc.VectorSubcoreMesh(core_axis_name="c", subcore_axis_name="s")

    @functools.partial(
        pl.kernel,
        mesh=mesh,
        compiler_params=pltpu.CompilerParams(use_tc_tiling_on_sc=False),
        out_type=jax.ShapeDtypeStruct((T, D), jnp.float32),
        scratch_types=[
            [pltpu.VMEM((C,), jnp.int32) for _ in range(2)],
            [pltpu.VMEM((C,), jnp.int32) for _ in range(2)],
            [pltpu.VMEM((C, D), jnp.float32) for _ in range(2)],
            [pltpu.SemaphoreType.DMA for _ in range(2)],
            [pltpu.SemaphoreType.DMA for _ in range(2)],
            [pltpu.SemaphoreType.DMA for _ in range(2)],
        ],
    )
    def lookup(x_hbm, m_hbm, table_hbm, out_hbm, idxs, msks, rows, sis, sgs,
               sos):
        wid = lax.axis_index("s") * NC + lax.axis_index("c")
        base_w = wid * per_w
        zi = jnp.zeros((_LANES,), jnp.int32)
        zf = jnp.zeros((_LANES,), jnp.float32)

        pltpu.async_copy(x_hbm.at[pl.ds(base_w, C)], idxs[0], sis[0])
        pltpu.async_copy(m_hbm.at[pl.ds(base_w, C)], msks[0], sis[0])

        def pair_body(p, carry):
            for b in (0, 1):
                ci = 2 * p + b
                base = base_w + ci * C
                idx_v, msk_v = idxs[b], msks[b]
                rows_v = rows[b]

                pltpu.make_async_copy(
                    x_hbm.at[pl.ds(base, C)], idx_v, sis[b]).wait()
                pltpu.make_async_copy(
                    m_hbm.at[pl.ds(base, C)], msk_v, sis[b]).wait()

                def remap_body(j, run_min):
                    sl = pl.ds(j * _LANES, _LANES)
                    xv = idx_v[sl]
                    mv = msk_v[sl]
                    run_min = jnp.minimum(run_min, jnp.minimum(xv, mv))
                    idx_v[sl] = jnp.where(xv == jnp.int32(_PAD_TOKEN),
                                          jnp.int32(_VOCAB_SIZE), xv)
                    return run_min

                run_min = lax.fori_loop(0, C // _LANES, remap_body, zi)
                lanes = [run_min[i] for i in range(_LANES)]
                while len(lanes) > 1:
                    lanes = [jnp.minimum(lanes[2 * i], lanes[2 * i + 1])
                             for i in range(len(lanes) // 2)]
                any_neg = lanes[0]

                @pl.when(ci + 1 < n_chunks)
                def _prefetch():
                    nb = base + C
                    pltpu.async_copy(
                        x_hbm.at[pl.ds(nb, C)], idxs[1 - b], sis[1 - b])
                    pltpu.async_copy(
                        m_hbm.at[pl.ds(nb, C)], msks[1 - b], sis[1 - b])

                @pl.when(ci >= 2)
                def _drain_out():
                    pltpu.make_async_copy(
                        rows_v, out_hbm.at[pl.ds(base, C)], sos[b]).wait()

                S = 4
                sub = C // S
                for s in range(S):
                    pltpu.async_copy(
                        table_hbm.at[idx_v.at[pl.ds(s * sub, sub)]],
                        rows_v.at[pl.ds(s * sub, sub)], sgs[b])
                for s in range(S):
                    pltpu.make_async_copy(
                        table_hbm.at[idx_v.at[pl.ds(s * sub, sub)]],
                        rows_v.at[pl.ds(s * sub, sub)], sgs[b]).wait()

                @pl.when(any_neg < 0)
                def _fix():
                    def fix_group(g, c2):
                        mv = msk_v[pl.ds(g * _LANES, _LANES)]
                        for r in range(_LANES):
                            ms = mv[r]
                            bad = jnp.logical_or(ms == _MASK_TOKEN,
                                                 ms == _PAD_TOKEN)

                            @pl.when(bad)
                            def _z(r=r):
                                row = g * _LANES + r
                                for h in range(D // _LANES):
                                    rows_v[row,
                                           pl.ds(h * _LANES, _LANES)] = zf

                        return c2

                    lax.fori_loop(0, C // _LANES, fix_group, 0)

                pltpu.async_copy(rows_v, out_hbm.at[pl.ds(base, C)], sos[b])
            return carry

        lax.fori_loop(0, n_chunks // 2, pair_body, 0)

        for b in (0, 1):
            base = base_w + (n_chunks - 2 + b) * C
            pltpu.make_async_copy(
                rows[b], out_hbm.at[pl.ds(base, C)], sos[b]).wait()

    return lookup


def kernel(x, mask, embedding):
    B, N = x.shape
    D = embedding.shape[1]
    T = B * N
    C = 1280
    out = _build_lookup(T, D, C)(
        x.reshape(T).astype(jnp.int32),
        mask.reshape(T).astype(jnp.int32),
        embedding,
    )
    return out.reshape(B, N, D)


# speedup vs baseline: 1.0099x; 1.0099x over previous
import functools

import jax
import jax.numpy as jnp
from jax import lax
from jax.experimental import pallas as pl
from jax.experimental.pallas import tpu as pltpu
from jax.experimental.pallas import tpu_sc as plsc

_VOCAB_SIZE = 1000000
_MASK_TOKEN = -1
_PAD_TOKEN = -2
_LANES = 16
_NSUB = 4


def _build_lookup(T, D, C):
    info = plsc.get_sparse_core_info()
    NC, NS = info.num_cores, info.num_subcores
    NW = NC * NS
    per_w = T // NW
    n_chunks = per_w // C
    S = C // _NSUB
    mesh = plsc.VectorSubcoreMesh(core_axis_name="c", subcore_axis_name="s")

    @functools.partial(
        pl.kernel,
        mesh=mesh,
        compiler_params=pltpu.CompilerParams(use_tc_tiling_on_sc=False),
        out_type=jax.ShapeDtypeStruct((T, D), jnp.float32),
        scratch_types=[
            [pltpu.VMEM((C,), jnp.int32) for _ in range(2)],
            [pltpu.VMEM((C,), jnp.int32) for _ in range(2)],
            [pltpu.VMEM((C,), jnp.int32) for _ in range(2)],
            [pltpu.VMEM((C, D), jnp.float32) for _ in range(2)],
            [pltpu.SemaphoreType.DMA for _ in range(2)],
            [[pltpu.SemaphoreType.DMA for _ in range(_NSUB)]
             for _ in range(2)],
            [pltpu.SemaphoreType.DMA for _ in range(2)],
        ],
    )
    def lookup(x_hbm, m_hbm, table_hbm, out_hbm, idxs, msks, rmps, rows,
               sis, sgs, sos):
        wid = lax.axis_index("s") * NC + lax.axis_index("c")
        base_w = wid * per_w
        zi = jnp.zeros((_LANES,), jnp.int32)
        zf = jnp.zeros((_LANES,), jnp.float32)

        pltpu.async_copy(x_hbm.at[pl.ds(base_w, C)], idxs[0], sis[0])
        pltpu.async_copy(m_hbm.at[pl.ds(base_w, C)], msks[0], sis[0])

        def drain(ci, b):
            base = base_w + ci * C
            for k in range(_NSUB):
                pltpu.make_async_copy(
                    table_hbm.at[rmps[b].at[pl.ds(k * S, S)]],
                    rows[b].at[pl.ds(k * S, S)], sgs[b][k]).wait()

            def run_fix(msk_v, rows_v):
                def fix_group(g, c2):
                    mv = msk_v[pl.ds(g * _LANES, _LANES)]
                    for r in range(_LANES):
                        ms = mv[r]
                        bad = jnp.logical_or(ms == _MASK_TOKEN,
                                             ms == _PAD_TOKEN)

                        @pl.when(bad)
                        def _z(r=r):
                            row = g * _LANES + r
                            for h in range(D // _LANES):
                                rows_v[row, pl.ds(h * _LANES, _LANES)] = zf

                    return c2

                lax.fori_loop(0, C // _LANES, fix_group, 0)

            def chunk_min(msk_v):
                def scan_body(j, run_min):
                    return jnp.minimum(run_min,
                                       msk_v[pl.ds(j * _LANES, _LANES)])

                run_min = lax.fori_loop(0, C // _LANES, scan_body, zi)
                lanes = [run_min[i] for i in range(_LANES)]
                while len(lanes) > 1:
                    lanes = [jnp.minimum(lanes[2 * i], lanes[2 * i + 1])
                             for i in range(len(lanes) // 2)]
                return lanes[0]

            @pl.when(chunk_min(msks[b]) < 0)
            def _fix():
                run_fix(msks[b], rows[b])

            pltpu.async_copy(rows[b], out_hbm.at[pl.ds(base, C)], sos[b])

        def pair_body(p, carry):
            for b in (0, 1):
                ci = 2 * p + b
                base = base_w + ci * C
                idx_v, msk_v, rmp_v = idxs[b], msks[b], rmps[b]

                pltpu.make_async_copy(
                    x_hbm.at[pl.ds(base, C)], idx_v, sis[b]).wait()
                pltpu.make_async_copy(
                    m_hbm.at[pl.ds(base, C)], msk_v, sis[b]).wait()

                def remap_body(j, _):
                    sl = pl.ds(j * _LANES, _LANES)
                    xv = idx_v[sl]
                    rmp_v[sl] = jnp.where(xv == jnp.int32(_PAD_TOKEN),
                                          jnp.int32(_VOCAB_SIZE), xv)
                    return 0

                lax.fori_loop(0, C // _LANES, remap_body, 0)

                @pl.when(ci >= 2)
                def _wb_done():
                    pb = base_w + (ci - 2) * C
                    pltpu.make_async_copy(
                        rows[b], out_hbm.at[pl.ds(pb, C)], sos[b]).wait()

                for k in range(_NSUB):
                    pltpu.async_copy(
                        table_hbm.at[rmp_v.at[pl.ds(k * S, S)]],
                        rows[b].at[pl.ds(k * S, S)], sgs[b][k])

                @pl.when(ci >= 1)
                def _drain_prev():
                    drain(ci - 1, 1 - b)

                @pl.when(ci + 1 < n_chunks)
                def _prefetch():
                    nb = base + C
                    pltpu.async_copy(
                        x_hbm.at[pl.ds(nb, C)], idxs[1 - b], sis[1 - b])
                    pltpu.async_copy(
                        m_hbm.at[pl.ds(nb, C)], msks[1 - b], sis[1 - b])

            return carry

        lax.fori_loop(0, n_chunks // 2, pair_body, 0)

        drain(n_chunks - 1, (n_chunks - 1) % 2)
        for b in (0, 1):
            base = base_w + (n_chunks - 2 + b) * C
            pltpu.make_async_copy(
                rows[b], out_hbm.at[pl.ds(base, C)], sos[b]).wait()

    return lookup


def kernel(x, mask, embedding):
    B, N = x.shape
    D = embedding.shape[1]
    T = B * N
    C = 1280
    out = _build_lookup(T, D, C)(
        x.reshape(T).astype(jnp.int32),
        mask.reshape(T).astype(jnp.int32),
        embedding,
    )
    return out.reshape(B, N, D)


# speedup vs baseline: 1.0101x; 1.0001x over previous
import functools

import jax
import jax.numpy as jnp
from jax import lax
from jax.experimental import pallas as pl
from jax.experimental.pallas import tpu as pltpu
from jax.experimental.pallas import tpu_sc as plsc

_VOCAB_SIZE = 1000000
_MASK_TOKEN = -1
_PAD_TOKEN = -2
_LANES = 16
_NSUB = 4


def _build_lookup(T, D, C):
    info = plsc.get_sparse_core_info()
    NC, NS = info.num_cores, info.num_subcores
    NW = NC * NS
    per_w = T // NW
    n_chunks = per_w // C
    S = C // _NSUB
    mesh = plsc.VectorSubcoreMesh(core_axis_name="c", subcore_axis_name="s")

    @functools.partial(
        pl.kernel,
        mesh=mesh,
        compiler_params=pltpu.CompilerParams(use_tc_tiling_on_sc=False),
        out_type=jax.ShapeDtypeStruct((T, D), jnp.float32),
        scratch_types=[
            [pltpu.VMEM((C,), jnp.int32) for _ in range(2)],
            [pltpu.VMEM((C,), jnp.int32) for _ in range(2)],
            [pltpu.VMEM((C,), jnp.int32) for _ in range(2)],
            [pltpu.VMEM((C, D), jnp.float32) for _ in range(2)],
            [pltpu.SemaphoreType.DMA for _ in range(2)],
            [[pltpu.SemaphoreType.DMA for _ in range(_NSUB)]
             for _ in range(2)],
            [pltpu.SemaphoreType.DMA for _ in range(2)],
        ],
    )
    def lookup(x_hbm, m_hbm, table_hbm, out_hbm, idxs, msks, rmps, rows,
               sis, sgs, sos):
        wid = lax.axis_index("s") * NC + lax.axis_index("c")
        base_w = wid * per_w
        zi = jnp.zeros((_LANES,), jnp.int32)
        zf = jnp.zeros((_LANES,), jnp.float32)

        pltpu.async_copy(x_hbm.at[pl.ds(base_w, C)], idxs[0], sis[0])
        pltpu.async_copy(m_hbm.at[pl.ds(base_w, C)], msks[0], sis[0])

        def drain(ci, b):
            base = base_w + ci * C
            for k in range(_NSUB):
                pltpu.make_async_copy(
                    table_hbm.at[rmps[b].at[pl.ds(k * S, S)]],
                    rows[b].at[pl.ds(k * S, S)], sgs[b][k]).wait()

            def run_fix(msk_v, rows_v):
                def fix_group(g, c2):
                    mv = msk_v[pl.ds(g * _LANES, _LANES)]
                    for r in range(_LANES):
                        ms = mv[r]
                        bad = jnp.logical_or(ms == _MASK_TOKEN,
                                             ms == _PAD_TOKEN)

                        @pl.when(bad)
                        def _z(r=r):
                            row = g * _LANES + r
                            for h in range(D // _LANES):
                                rows_v[row, pl.ds(h * _LANES, _LANES)] = zf

                    return c2

                lax.fori_loop(0, C // _LANES, fix_group, 0)

            def chunk_min(msk_v):
                def scan_body(j, run_min):
                    return jnp.minimum(run_min,
                                       msk_v[pl.ds(j * _LANES, _LANES)])

                run_min = lax.fori_loop(0, C // _LANES, scan_body, zi)
                lanes = [run_min[i] for i in range(_LANES)]
                while len(lanes) > 1:
                    lanes = [jnp.minimum(lanes[2 * i], lanes[2 * i + 1])
                             for i in range(len(lanes) // 2)]
                return lanes[0]

            @pl.when(chunk_min(msks[b]) < 0)
            def _fix():
                run_fix(msks[b], rows[b])

            pltpu.async_copy(rows[b], out_hbm.at[pl.ds(base, C)], sos[b])

        def pair_body(p, carry):
            for b in (0, 1):
                ci = 2 * p + b
                base = base_w + ci * C
                idx_v, msk_v, rmp_v = idxs[b], msks[b], rmps[b]

                pltpu.make_async_copy(
                    x_hbm.at[pl.ds(base, C)], idx_v, sis[b]).wait()
                pltpu.make_async_copy(
                    m_hbm.at[pl.ds(base, C)], msk_v, sis[b]).wait()

                def remap_body(j, _):
                    sl = pl.ds(j * _LANES, _LANES)
                    xv = idx_v[sl]
                    rmp_v[sl] = jnp.where(xv == jnp.int32(_PAD_TOKEN),
                                          jnp.int32(_VOCAB_SIZE), xv)
                    return 0

                lax.fori_loop(0, C // _LANES, remap_body, 0)

                @pl.when(ci >= 2)
                def _wb_done():
                    pb = base_w + (ci - 2) * C
                    pltpu.make_async_copy(
                        rows[b], out_hbm.at[pl.ds(pb, C)], sos[b]).wait()

                for k in range(_NSUB):
                    pltpu.async_copy(
                        table_hbm.at[rmp_v.at[pl.ds(k * S, S)]],
                        rows[b].at[pl.ds(k * S, S)], sgs[b][k])

                @pl.when(ci >= 1)
                def _drain_prev():
                    drain(ci - 1, 1 - b)

                @pl.when(ci + 1 < n_chunks)
                def _prefetch():
                    nb = base + C
                    pltpu.async_copy(
                        x_hbm.at[pl.ds(nb, C)], idxs[1 - b], sis[1 - b])
                    pltpu.async_copy(
                        m_hbm.at[pl.ds(nb, C)], msks[1 - b], sis[1 - b])

            return carry

        lax.fori_loop(0, n_chunks // 2, pair_body, 0)

        drain(n_chunks - 1, (n_chunks - 1) % 2)
        for b in (0, 1):
            base = base_w + (n_chunks - 2 + b) * C
            pltpu.make_async_copy(
                rows[b], out_hbm.at[pl.ds(base, C)], sos[b]).wait()

    return lookup


def kernel(x, mask, embedding):
    B, N = x.shape
    D = embedding.shape[1]
    T = B * N
    C = 1600
    out = _build_lookup(T, D, C)(
        x.reshape(T).astype(jnp.int32),
        mask.reshape(T).astype(jnp.int32),
        embedding,
    )
    return out.reshape(B, N, D)
